# initial kernel scaffold (unmeasured)
import jax
import jax.numpy as jnp
from jax import lax
from jax.experimental import pallas as pl
from jax.experimental.pallas import tpu as pltpu

D = 4096
EPS = 1e-6


def _exchange_body(p_ref, out_ref, send_sem, recv_sem):
    mx = lax.axis_index("x")
    my = lax.axis_index("y")
    mz = lax.axis_index("z")
    partner = (mx, my, 1 - mz)

    bar = pltpu.get_barrier_semaphore()
    pl.semaphore_signal(
        bar, inc=1, device_id=partner, device_id_type=pl.DeviceIdType.MESH
    )
    pl.semaphore_wait(bar, 1)

    rdma = pltpu.make_async_remote_copy(
        src_ref=p_ref,
        dst_ref=out_ref,
        send_sem=send_sem,
        recv_sem=recv_sem,
        device_id=partner,
        device_id_type=pl.DeviceIdType.MESH,
    )
    rdma.start()
    rdma.wait()


def _ln_body(a_ref, b_ref, r_ref, g_ref, o_ref):
    y = a_ref[...] + b_ref[...] + r_ref[...]
    ms = jnp.mean(y * y, axis=-1, keepdims=True)
    o_ref[...] = y * lax.rsqrt(ms + EPS) * g_ref[...]


def kernel(partial, resid, gamma):
    p = partial.reshape(D, D)
    g = gamma.reshape(1, D)

    other = pl.pallas_call(
        _exchange_body,
        out_shape=jax.ShapeDtypeStruct((D, D), jnp.float32),
        in_specs=[pl.BlockSpec(memory_space=pltpu.ANY)],
        out_specs=pl.BlockSpec(memory_space=pltpu.ANY),
        scratch_shapes=[pltpu.SemaphoreType.DMA, pltpu.SemaphoreType.DMA],
        compiler_params=pltpu.CompilerParams(collective_id=0),
    )(p)

    C = 512
    out = pl.pallas_call(
        _ln_body,
        out_shape=jax.ShapeDtypeStruct((D, D), jnp.float32),
        grid=(D // C,),
        in_specs=[
            pl.BlockSpec((C, D), lambda i: (i, 0)),
            pl.BlockSpec((C, D), lambda i: (i, 0)),
            pl.BlockSpec((C, D), lambda i: (i, 0)),
            pl.BlockSpec((1, D), lambda i: (0, 0)),
        ],
        out_specs=pl.BlockSpec((C, D), lambda i: (i, 0)),
        compiler_params=pltpu.CompilerParams(
            dimension_semantics=("arbitrary",)
        ),
    )(p, other, resid, g)
    return out


# baseline (device time: 807352 ns/iter reference)
import jax
import jax.numpy as jnp
from jax import lax
from jax.experimental import pallas as pl
from jax.experimental.pallas import tpu as pltpu

D = 4096
EPS = 1e-6


def _exchange_body(p_ref, out_ref, send_sem, recv_sem):
    mx = lax.axis_index("x")
    my = lax.axis_index("y")
    mz = lax.axis_index("z")
    partner = (mx, my, 1 - mz)

    bar = pltpu.get_barrier_semaphore()
    pl.semaphore_signal(
        bar, inc=1, device_id=partner, device_id_type=pl.DeviceIdType.MESH
    )
    pl.semaphore_wait(bar, 1)

    rdma = pltpu.make_async_remote_copy(
        src_ref=p_ref,
        dst_ref=out_ref,
        send_sem=send_sem,
        recv_sem=recv_sem,
        device_id=partner,
        device_id_type=pl.DeviceIdType.MESH,
    )
    rdma.start()
    rdma.wait()


def _ln_body(a_ref, b_ref, r_ref, g_ref, o_ref):
    y = a_ref[...] + b_ref[...] + r_ref[...]
    ms = jnp.mean(y * y, axis=-1, keepdims=True)
    o_ref[...] = y * lax.rsqrt(ms + EPS) * g_ref[...]


def kernel(partial, resid, gamma):
    p = partial.reshape(D, D)
    g = gamma.reshape(1, D)

    other = pl.pallas_call(
        _exchange_body,
        out_shape=jax.ShapeDtypeStruct((D, D), jnp.float32),
        in_specs=[pl.BlockSpec(memory_space=pl.ANY)],
        out_specs=pl.BlockSpec(memory_space=pl.ANY),
        scratch_shapes=[pltpu.SemaphoreType.DMA, pltpu.SemaphoreType.DMA],
        compiler_params=pltpu.CompilerParams(collective_id=0),
    )(p)

    C = 128
    out = pl.pallas_call(
        _ln_body,
        out_shape=jax.ShapeDtypeStruct((D, D), jnp.float32),
        grid=(D // C,),
        in_specs=[
            pl.BlockSpec((C, D), lambda i: (i, 0)),
            pl.BlockSpec((C, D), lambda i: (i, 0)),
            pl.BlockSpec((C, D), lambda i: (i, 0)),
            pl.BlockSpec((1, D), lambda i: (0, 0)),
        ],
        out_specs=pl.BlockSpec((C, D), lambda i: (i, 0)),
        compiler_params=pltpu.CompilerParams(
            dimension_semantics=("arbitrary",)
        ),
    )(p, other, resid, g)
    return out


# device time: 441981 ns/iter; 1.8267x vs baseline; 1.8267x over previous
import jax
import jax.numpy as jnp
from jax import lax
from jax.experimental import pallas as pl
from jax.experimental.pallas import tpu as pltpu

D = 4096
HALF = D // 2
C = 128
G = HALF // C
EPS = 1e-6
_MESH = pl.DeviceIdType.MESH


def _body(p_ref, r_ref, g_ref, out_ref, zbuf_ref, xbuf_ref,
          a_ref, b_ref, c_ref, o_ref,
          zs, zr, xs, xr, in_sems, out_sems):
    mx = lax.axis_index("x")
    my = lax.axis_index("y")
    mz = lax.axis_index("z")
    zpartner = (mx, my, 1 - mz)
    xneighbor = (1 - mx, my, mz)

    bar = pltpu.get_barrier_semaphore()
    pl.semaphore_signal(bar, inc=1, device_id=zpartner, device_id_type=_MESH)
    pl.semaphore_signal(bar, inc=1, device_id=xneighbor, device_id_type=_MESH)
    pl.semaphore_wait(bar, 2)

    zb = HALF * mx
    xb = HALF * (1 - mx)

    rdma_z = []
    for c in range(G):
        rz = pltpu.make_async_remote_copy(
            src_ref=p_ref.at[pl.ds(zb + c * C, C)],
            dst_ref=zbuf_ref.at[pl.ds(c * C, C)],
            send_sem=zs.at[c],
            recv_sem=zr.at[c],
            device_id=zpartner,
            device_id_type=_MESH,
        )
        rz.start()
        rdma_z.append(rz)

    rdma_x = [None] * G

    jobs = [("z", 0)]
    for c in range(1, G):
        jobs += [("z", c), ("x", c - 1)]
    jobs.append(("x", G - 1))

    def wait_arrival(kind, c):
        if kind == "z":
            rdma_z[c].wait_recv()
            rx = pltpu.make_async_remote_copy(
                src_ref=zbuf_ref.at[pl.ds(c * C, C)],
                dst_ref=xbuf_ref.at[pl.ds(c * C, C)],
                send_sem=xs.at[c],
                recv_sem=xr.at[c],
                device_id=xneighbor,
                device_id_type=_MESH,
            )
            rx.start()
            rdma_x[c] = rx
        else:
            rdma_x[c].wait_recv()

    def start_in(j):
        kind, c = jobs[j]
        s = j % 2
        base = (zb if kind == "z" else xb) + c * C
        buf = zbuf_ref if kind == "z" else xbuf_ref
        cps = [
            pltpu.make_async_copy(
                p_ref.at[pl.ds(base, C)], a_ref.at[s], in_sems.at[s, 0]),
            pltpu.make_async_copy(
                buf.at[pl.ds(c * C, C)], b_ref.at[s], in_sems.at[s, 1]),
            pltpu.make_async_copy(
                r_ref.at[pl.ds(base, C)], c_ref.at[s], in_sems.at[s, 2]),
        ]
        for cp in cps:
            cp.start()
        return cps

    pending_in = {}
    pending_out = {}
    wait_arrival(*jobs[0])
    pending_in[0] = start_in(0)
    for j in range(len(jobs)):
        if j + 1 < len(jobs):
            wait_arrival(*jobs[j + 1])
            pending_in[j + 1] = start_in(j + 1)
        for cp in pending_in.pop(j):
            cp.wait()
        s = j % 2
        if s in pending_out:
            pending_out.pop(s).wait()
        kind, c = jobs[j]
        y = a_ref[s] + b_ref[s] + c_ref[s]
        ms = jnp.mean(y * y, axis=-1, keepdims=True)
        o_ref[s] = y * lax.rsqrt(ms + EPS) * g_ref[...]
        base = (zb if kind == "z" else xb) + c * C
        cpo = pltpu.make_async_copy(
            o_ref.at[s], out_ref.at[pl.ds(base, C)], out_sems.at[s])
        cpo.start()
        pending_out[s] = cpo

    for cpo in pending_out.values():
        cpo.wait()
    for c in range(G):
        rdma_z[c].wait_send()
        rdma_x[c].wait_send()


def kernel(partial, resid, gamma):
    p = partial.reshape(D, D)
    g = gamma.reshape(1, D)
    out, _, _ = pl.pallas_call(
        _body,
        out_shape=[
            jax.ShapeDtypeStruct((D, D), jnp.float32),
            jax.ShapeDtypeStruct((HALF, D), jnp.float32),
            jax.ShapeDtypeStruct((HALF, D), jnp.float32),
        ],
        in_specs=[
            pl.BlockSpec(memory_space=pl.ANY),
            pl.BlockSpec(memory_space=pl.ANY),
            pl.BlockSpec(memory_space=pltpu.MemorySpace.VMEM),
        ],
        out_specs=[pl.BlockSpec(memory_space=pl.ANY)] * 3,
        scratch_shapes=[
            pltpu.VMEM((2, C, D), jnp.float32),
            pltpu.VMEM((2, C, D), jnp.float32),
            pltpu.VMEM((2, C, D), jnp.float32),
            pltpu.VMEM((2, C, D), jnp.float32),
            pltpu.SemaphoreType.DMA((G,)),
            pltpu.SemaphoreType.DMA((G,)),
            pltpu.SemaphoreType.DMA((G,)),
            pltpu.SemaphoreType.DMA((G,)),
            pltpu.SemaphoreType.DMA((2, 3)),
            pltpu.SemaphoreType.DMA((2,)),
        ],
        compiler_params=pltpu.CompilerParams(collective_id=0),
    )(p, resid, g)
    return out
